# baseline (speedup 1.0000x reference)
import functools

import jax
import jax.numpy as jnp
from jax import lax
from jax.experimental import pallas as pl
from jax.experimental.pallas import tpu as pltpu
from jax.experimental.pallas import tpu_sc as plsc

B = 16384
CONT = 13
NCAT = 26
VOCAB = 100000
D = 64

NC = 2
NS = 16
NW = NC * NS
CAT_ROWS = NCAT * D
CONT_ROWS = CONT * D
CAT_PW = CAT_ROWS // NW
CONT_PW = CONT_ROWS // NW
CHUNK = 1024
NCHUNK = B // CHUNK
NSLOT = 6
CQ = 8 * CHUNK


def _sc_body(xcat_hbm, xcont_hbm, tab_hbm, ctab_hbm, out_hbm,
             row_v, col_v, colq_v, ring_v, ctab_v, rsem, ssem, qsem, csem):
  w = lax.axis_index("s") * NC + lax.axis_index("c")

  def drain_stores():
    pltpu.make_async_copy(
        tab_hbm.at[0, 0, pl.ds(0, NSLOT * CHUNK)], ring_v, ssem).wait()

  def fetch_colq(kk):
    v2 = w * CONT_PW + kk // 2
    pltpu.async_copy(
        xcont_hbm.at[v2 // D, pl.ds((kk % 2) * CQ, CQ)], colq_v, qsem)

  u0 = w * CAT_PW
  pltpu.async_copy(xcat_hbm.at[u0 // D], col_v, csem)
  fetch_colq(0)
  pltpu.async_copy(tab_hbm.at[u0 // D, u0 % D], row_v, rsem)

  def item(k, carry):
    f_prev, c_prev = carry
    u = w * CAT_PW + k
    f = u // D
    d = u % D

    @pl.when(k > 0)
    def _():
      drain_stores()
    @pl.when(f != f_prev)
    def _():
      pltpu.make_async_copy(xcat_hbm.at[f], col_v, csem).wait()

    v = w * CONT_PW + k // 2
    cc = v // D
    dc = v % D
    h = k % 2
    @pl.when(cc != c_prev)
    def _():
      pltpu.sync_copy(ctab_hbm.at[cc], ctab_v)
    s_vec = plsc.load_gather(ctab_v, [jnp.full((16,), dc, dtype=jnp.int32)])
    pltpu.make_async_copy(
        xcont_hbm.at[cc, pl.ds(h * CQ, CQ)], colq_v, qsem).wait()

    sds = []
    for i in range(NCHUNK // 2):
      c_abs = h * (NCHUNK // 2) + i
      if i >= NSLOT:
        sds[i - NSLOT].wait()

      @plsc.parallel_loop(0, CHUNK, step=16, unroll=8)
      def _(off):
        xv = plsc.bitcast(colq_v[pl.ds(i * CHUNK + off, 16)], jnp.float32)
        ring_v[pl.ds((i % NSLOT) * CHUNK + off, 16)] = xv * s_vec

      sds.append(pltpu.async_copy(
          ring_v.at[pl.ds((i % NSLOT) * CHUNK, CHUNK)],
          out_hbm.at[cc, dc, pl.ds(c_abs * CHUNK, CHUNK)], ssem))

    pltpu.make_async_copy(tab_hbm.at[f, d], row_v, rsem).wait()

    for c in range(NCHUNK):
      j = NCHUNK // 2 + c
      slot = j % NSLOT
      sds[j - NSLOT].wait()

      @plsc.parallel_loop(0, CHUNK, step=16, unroll=8)
      def _(off):
        idx = col_v[pl.ds(c * CHUNK + off, 16)]
        ring_v[pl.ds(slot * CHUNK + off, 16)] = plsc.load_gather(
            row_v, [idx])

      sds.append(pltpu.async_copy(
          ring_v.at[pl.ds(slot * CHUNK, CHUNK)],
          out_hbm.at[CONT + f, d, pl.ds(c * CHUNK, CHUNK)], ssem))

    @pl.when(k < CAT_PW - 1)
    def _():
      u2 = u + 1
      f2 = u2 // D
      @pl.when(f2 != f)
      def _():
        pltpu.async_copy(xcat_hbm.at[f2], col_v, csem)
      fetch_colq(k + 1)
      pltpu.async_copy(tab_hbm.at[f2, u2 % D], row_v, rsem)

    return f, cc

  lax.fori_loop(0, CAT_PW, item, (jnp.int32(-1), jnp.int32(-1)))
  drain_stores()


@jax.jit
def kernel(x_cont, x_cat, cat_tables, cont_table):
  f32 = jnp.float32
  i32 = jnp.int32
  tabT = jnp.transpose(cat_tables, (0, 2, 1))
  xcatT = x_cat.astype(i32).T
  xcontT = lax.bitcast_convert_type(x_cont.T, i32)

  mesh = plsc.VectorSubcoreMesh(core_axis_name="c", subcore_axis_name="s",
                                num_cores=NC, num_subcores=NS)
  out = pl.kernel(
      _sc_body,
      out_type=jax.ShapeDtypeStruct((CONT + NCAT, D, B), f32),
      mesh=mesh,
      compiler_params=pltpu.CompilerParams(
          use_tc_tiling_on_sc=True, needs_layout_passes=False),
      scratch_types=[
          pltpu.VMEM((VOCAB,), f32),
          pltpu.VMEM((B,), i32),
          pltpu.VMEM((CQ,), i32),
          pltpu.VMEM((NSLOT * CHUNK,), f32),
          pltpu.VMEM((D,), f32),
          pltpu.SemaphoreType.DMA,
          pltpu.SemaphoreType.DMA,
          pltpu.SemaphoreType.DMA,
          pltpu.SemaphoreType.DMA,
      ],
  )(xcatT, xcontT, tabT, cont_table)
  return jnp.transpose(out, (2, 0, 1))


# speedup vs baseline: 1.0281x; 1.0281x over previous
import functools

import jax
import jax.numpy as jnp
from jax import lax
from jax.experimental import pallas as pl
from jax.experimental.pallas import tpu as pltpu
from jax.experimental.pallas import tpu_sc as plsc

B = 16384
CONT = 13
NCAT = 26
VOCAB = 100000
D = 64

NC = 2
NS = 16
NW = NC * NS
CAT_ROWS = NCAT * D
CONT_ROWS = CONT * D
CAT_PW = CAT_ROWS // NW
CONT_PW = CONT_ROWS // NW
CHUNK = 4096
NCHUNK = B // CHUNK
NSLOT = 2


def _sc_body(xcat_hbm, xcont_hbm, tab_hbm, ctab_hbm, out_hbm,
             row_v, col_v, ring_v, ctab_v, rsem, csem, ssem):
  w = lax.axis_index("s") * NC + lax.axis_index("c")

  def drain_stores():
    pltpu.make_async_copy(
        tab_hbm.at[0, 0, pl.ds(0, NSLOT * CHUNK)], ring_v, ssem).wait()

  def emit_row(j, d, gen_chunk):
    sds = []
    for c in range(NCHUNK):
      slot = c % NSLOT
      if c >= NSLOT:
        sds[c - NSLOT].wait()
      gen_chunk(c, slot)
      sds.append(pltpu.async_copy(
          ring_v.at[pl.ds(slot * CHUNK, CHUNK)],
          out_hbm.at[j, d, pl.ds(c * CHUNK, CHUNK)], ssem))

  def cat_item(k, f_prev):
    u = w * CAT_PW + k
    f = u // D
    d = u % D
    row_copy = pltpu.async_copy(tab_hbm.at[f, d], row_v, rsem)
    @pl.when(k > 0)
    def _():
      drain_stores()
    @pl.when(f != f_prev)
    def _():
      pltpu.sync_copy(xcat_hbm.at[f], col_v)
    row_copy.wait()

    def gen_chunk(c, slot):
      @plsc.parallel_loop(0, CHUNK, step=16, unroll=8)
      def _(off):
        idx = col_v[pl.ds(c * CHUNK + off, 16)]
        ring_v[pl.ds(slot * CHUNK + off, 16)] = plsc.load_gather(
            row_v, [idx])

    emit_row(CONT + f, d, gen_chunk)
    return f

  lax.fori_loop(0, CAT_PW, cat_item, jnp.int32(-1))

  def cont_item(m, c_prev):
    v = w * CONT_PW + m
    cc = v // D
    d = v % D
    drain_stores()
    @pl.when(cc != c_prev)
    def _():
      pltpu.sync_copy(xcont_hbm.at[cc], col_v)
      pltpu.sync_copy(ctab_hbm.at[cc], ctab_v)
    s_vec = plsc.load_gather(ctab_v, [jnp.full((16,), d, dtype=jnp.int32)])

    def gen_chunk(c, slot):
      @plsc.parallel_loop(0, CHUNK, step=16, unroll=8)
      def _(off):
        xv = plsc.bitcast(col_v[pl.ds(c * CHUNK + off, 16)], jnp.float32)
        ring_v[pl.ds(slot * CHUNK + off, 16)] = xv * s_vec

    emit_row(cc, d, gen_chunk)
    return cc

  lax.fori_loop(0, CONT_PW, cont_item, jnp.int32(-1))
  drain_stores()


@jax.jit
def kernel(x_cont, x_cat, cat_tables, cont_table):
  f32 = jnp.float32
  i32 = jnp.int32
  tabT = jnp.transpose(cat_tables, (0, 2, 1))
  xcatT = x_cat.astype(i32).T
  xcontT = lax.bitcast_convert_type(x_cont.T, i32)

  mesh = plsc.VectorSubcoreMesh(core_axis_name="c", subcore_axis_name="s",
                                num_cores=NC, num_subcores=NS)
  out = pl.kernel(
      _sc_body,
      out_type=jax.ShapeDtypeStruct((CONT + NCAT, D, B), f32),
      mesh=mesh,
      compiler_params=pltpu.CompilerParams(
          use_tc_tiling_on_sc=True, needs_layout_passes=False),
      scratch_types=[
          pltpu.VMEM((VOCAB,), f32),
          pltpu.VMEM((B,), i32),
          pltpu.VMEM((NSLOT * CHUNK,), f32),
          pltpu.VMEM((D,), f32),
          pltpu.SemaphoreType.DMA,
          pltpu.SemaphoreType.DMA,
          pltpu.SemaphoreType.DMA,
      ],
  )(xcatT, xcontT, tabT, cont_table)
  return jnp.transpose(out, (2, 0, 1))
